# Initial kernel scaffold; baseline (speedup 1.0000x reference)
import jax
import jax.numpy as jnp
from jax.experimental import pallas as pl


def kernel(inputs):
    raise NotImplementedError("write your pallas kernel here")


# speedup vs baseline: 21.7179x; 21.7179x over previous
import functools

import jax
import jax.numpy as jnp
from jax import lax
from jax.experimental import pallas as pl
from jax.experimental.pallas import tpu as pltpu
from jax.experimental.pallas import tpu_sc as plsc

_NPOINTS = 2048
_NSAMPLE = 32


def _sc_gather(table, idx2d):
    irows, lanes = idx2d.shape
    assert lanes == 128
    rows = irows * lanes
    channels = table.shape[1]

    info = plsc.get_sparse_core_info()
    num_workers = info.num_cores * info.num_subcores
    irows_pw = irows // num_workers
    ki = 16
    nblocks = irows_pw // ki
    block_rows = ki * lanes

    mesh = plsc.VectorSubcoreMesh(core_axis_name="c", subcore_axis_name="s")

    @functools.partial(
        pl.kernel,
        out_type=jax.ShapeDtypeStruct((rows, channels), table.dtype),
        mesh=mesh,
        scratch_types=[
            pltpu.VMEM((irows_pw, lanes), jnp.int32),
            pltpu.VMEM((block_rows, channels), table.dtype),
            pltpu.SemaphoreType.DMA,
        ],
        compiler_params=pltpu.CompilerParams(use_tc_tiling_on_sc=False),
    )
    def gather_kernel(table_hbm, idx_hbm, out_hbm, idx_v, rows_v, sem):
        wid = lax.axis_index("s") * info.num_cores + lax.axis_index("c")
        irow0 = wid * irows_pw
        pltpu.sync_copy(idx_hbm.at[pl.ds(irow0, irows_pw)], idx_v)

        def block(nb, carry):
            copies = []
            for j in range(ki):
                copies.append(pltpu.async_copy(
                    table_hbm.at[idx_v.at[nb * ki + j]],
                    rows_v.at[pl.ds(j * lanes, lanes)],
                    sem))
            for cp in copies:
                cp.wait()
            row0 = wid * (irows_pw * lanes) + nb * block_rows
            pltpu.sync_copy(rows_v, out_hbm.at[pl.ds(row0, block_rows)])
            return carry

        lax.fori_loop(0, nblocks, block, 0)

    return gather_kernel(table, idx2d)


def kernel(inputs):
    b, n, c = inputs.shape
    key = jax.random.key(42)
    k1, k2 = jax.random.split(key)
    group_indices = jax.random.randint(
        k2, (b, _NPOINTS, _NSAMPLE), 0, n, dtype=jnp.int32)
    flat_idx = group_indices.reshape(b, _NPOINTS * _NSAMPLE)
    gidx = flat_idx + (jnp.arange(b, dtype=jnp.int32) * n)[:, None]
    out = _sc_gather(inputs.reshape(b * n, c), gidx.reshape(-1, 128))
    return out.reshape(b, _NPOINTS, _NSAMPLE, c)


# speedup vs baseline: 56.4846x; 2.6008x over previous
import functools

import jax
import jax.numpy as jnp
from jax import lax
from jax.experimental import pallas as pl
from jax.experimental.pallas import tpu as pltpu
from jax.experimental.pallas import tpu_sc as plsc

_NPOINTS = 2048
_NSAMPLE = 32


def _sc_gather_tiled(w, idx2d, b_dim, s_dim):
    mesh = plsc.VectorSubcoreMesh(core_axis_name="c", subcore_axis_name="s")
    info = plsc.get_sparse_core_info()
    num_workers = info.num_cores * info.num_subcores

    @functools.partial(
        pl.kernel,
        out_type=jax.ShapeDtypeStruct((b_dim, s_dim, 2, 16, 8, 128), w.dtype),
        mesh=mesh,
        scratch_types=[
            pltpu.VMEM((128, 4, 128), jnp.float32),
            pltpu.VMEM((16, 128), jnp.int32),
            pltpu.VMEM((16, 4, 128), jnp.float32),
        ],
        compiler_params=pltpu.CompilerParams(
            use_tc_tiling_on_sc=False, needs_layout_passes=False),
    )
    def gather_kernel(w_hbm, idx_hbm, out_hbm, table_v, idx_v, out_v):
        wid = lax.axis_index("s") * info.num_cores + lax.axis_index("c")
        b = wid // 2
        ct = wid % 2

        for h in range(2):
            pltpu.sync_copy(w_hbm.at[b, ct, :, pl.ds(4 * h, 4), :],
                            table_v)

            def sample(s, carry):
                pltpu.sync_copy(idx_hbm.at[pl.ds((b * s_dim + s) * 16, 16)],
                                idx_v)

                def ptile(pt, c2):
                    for gg in range(8):
                        n = idx_v[pt, pl.ds(gg * 16, 16)]
                        nt = lax.shift_right_logical(n, 7)
                        nj = lax.bitwise_and(n, 127)
                        for ci_ in range(4):
                            ci_arr = jnp.full((16,), ci_, jnp.int32)
                            vals = plsc.load_gather(
                                table_v, [nt, ci_arr, nj])
                            out_v[pt, ci_, pl.ds(gg * 16, 16)] = vals
                    return c2

                lax.fori_loop(0, 16, ptile, 0)
                pltpu.sync_copy(
                    out_v,
                    out_hbm.at[b, s, ct, :, pl.ds(4 * h, 4), :])
                return carry

            lax.fori_loop(0, s_dim, sample, 0)

    return gather_kernel(w, idx2d)


def kernel(inputs):
    b, n, c = inputs.shape
    key = jax.random.key(42)
    k1, k2 = jax.random.split(key)
    group_indices = jax.random.randint(
        k2, (b, _NPOINTS, _NSAMPLE), 0, n, dtype=jnp.int32)
    idx2d = group_indices.transpose(0, 2, 1).reshape(-1, 128)
    w = inputs.transpose(0, 2, 1).reshape(b, 2, 8, 128, 128)
    w = w.transpose(0, 1, 3, 2, 4)
    o6 = _sc_gather_tiled(w, idx2d, b, _NSAMPLE)
    out = o6.transpose(0, 1, 2, 4, 3, 5).reshape(b, _NSAMPLE, c, _NPOINTS)
    return out.transpose(0, 3, 1, 2)


# speedup vs baseline: 70.1565x; 1.2420x over previous
import functools

import jax
import jax.numpy as jnp
from jax import lax
from jax.experimental import pallas as pl
from jax.experimental.pallas import tpu as pltpu
from jax.experimental.pallas import tpu_sc as plsc

_NPOINTS = 2048
_NSAMPLE = 32


def _sc_gather_tiled(w, idx2d, b_dim, s_dim):
    mesh = plsc.VectorSubcoreMesh(core_axis_name="c", subcore_axis_name="s")
    info = plsc.get_sparse_core_info()

    sq = 16
    nq = s_dim // sq

    @functools.partial(
        pl.kernel,
        out_type=jax.ShapeDtypeStruct((b_dim, s_dim, 2, 16, 8, 128), w.dtype),
        mesh=mesh,
        scratch_types=[
            pltpu.VMEM((128, 4, 128), jnp.float32),
            pltpu.VMEM((16 * sq, 128), jnp.int32),
            pltpu.VMEM((16, 4, 128), jnp.float32),
            pltpu.VMEM((16, 4, 128), jnp.float32),
            pltpu.SemaphoreType.DMA,
            pltpu.SemaphoreType.DMA,
        ],
        compiler_params=pltpu.CompilerParams(
            use_tc_tiling_on_sc=False, needs_layout_passes=False),
    )
    def gather_kernel(w_hbm, idx_hbm, out_hbm, table_v, idx_v,
                      out_v0, out_v1, so0, so1):
        wid = lax.axis_index("s") * info.num_cores + lax.axis_index("c")
        b = wid // 2
        ct = wid % 2
        out_bufs = (out_v0, out_v1)
        sems = (so0, so1)

        for h in range(2):
            pltpu.sync_copy(w_hbm.at[b, ct, :, pl.ds(4 * h, 4), :], table_v)

            for q in range(nq):
                pltpu.sync_copy(
                    idx_hbm.at[pl.ds((b * s_dim + q * sq) * 16, 16 * sq)],
                    idx_v)

                def pair(t, carry, q=q, h=h):
                    for p in range(2):
                        sl = 2 * t + p
                        s = q * sq + sl

                        @pl.when(jnp.logical_or(sl >= 2, (q + h) > 0))
                        def _():
                            pltpu.make_async_copy(
                                out_bufs[p],
                                out_hbm.at[b, s, ct, :, pl.ds(4 * h, 4), :],
                                sems[p]).wait()

                        def ptile(pt, c2, p=p, sl=sl):
                            for gg in range(8):
                                n = idx_v[sl * 16 + pt, pl.ds(gg * 16, 16)]
                                nt = lax.shift_right_logical(n, 7)
                                nj = lax.bitwise_and(n, 127)
                                for ci_ in range(4):
                                    ci_arr = jnp.full((16,), ci_, jnp.int32)
                                    vals = plsc.load_gather(
                                        table_v, [nt, ci_arr, nj])
                                    out_bufs[p][
                                        pt, ci_, pl.ds(gg * 16, 16)] = vals
                            return c2

                        lax.fori_loop(0, 16, ptile, 0)
                        pltpu.async_copy(
                            out_bufs[p],
                            out_hbm.at[b, s, ct, :, pl.ds(4 * h, 4), :],
                            sems[p])
                    return carry

                lax.fori_loop(0, sq // 2, pair, 0)

        for p in range(2):
            pltpu.make_async_copy(
                out_bufs[p],
                out_hbm.at[b, 0, ct, :, pl.ds(4, 4), :],
                sems[p]).wait()

    return gather_kernel(w, idx2d)


def kernel(inputs):
    b, n, c = inputs.shape
    key = jax.random.key(42)
    k1, k2 = jax.random.split(key)
    group_indices = jax.random.randint(
        k2, (b, _NPOINTS, _NSAMPLE), 0, n, dtype=jnp.int32)
    idx2d = group_indices.transpose(0, 2, 1).reshape(-1, 128)
    w = inputs.transpose(0, 2, 1).reshape(b, 2, 8, 128, 128)
    w = w.transpose(0, 1, 3, 2, 4)
    o6 = _sc_gather_tiled(w, idx2d, b, _NSAMPLE)
    out = o6.transpose(0, 1, 2, 4, 3, 5).reshape(b, _NSAMPLE, c, _NPOINTS)
    return out.transpose(0, 3, 1, 2)


# speedup vs baseline: 122.4706x; 1.7457x over previous
import functools

import jax
import jax.numpy as jnp
from jax import lax
from jax.experimental import pallas as pl
from jax.experimental.pallas import tpu as pltpu
from jax.experimental.pallas import tpu_sc as plsc

_NPOINTS = 2048
_NSAMPLE = 32


def _sc_gather_tiled(w, idx2d, b_dim, s_dim):
    mesh = plsc.VectorSubcoreMesh(core_axis_name="c", subcore_axis_name="s")
    info = plsc.get_sparse_core_info()

    sq = 16
    nq = s_dim // sq

    @functools.partial(
        pl.kernel,
        out_type=jax.ShapeDtypeStruct((b_dim, s_dim, 2, 16, 8, 128), w.dtype),
        mesh=mesh,
        scratch_types=[
            pltpu.VMEM((128, 4, 128), jnp.float32),
            pltpu.VMEM((16 * sq, 128), jnp.int32),
            pltpu.VMEM((16, 4, 128), jnp.float32),
            pltpu.VMEM((16, 4, 128), jnp.float32),
            pltpu.SemaphoreType.DMA,
            pltpu.SemaphoreType.DMA,
        ],
        compiler_params=pltpu.CompilerParams(
            use_tc_tiling_on_sc=False, needs_layout_passes=False),
    )
    def gather_kernel(w_hbm, idx_hbm, out_hbm, table_v, idx_v,
                      out_v0, out_v1, so0, so1):
        wid = lax.axis_index("s") * info.num_cores + lax.axis_index("c")
        b = wid // 2
        ct = wid % 2
        out_bufs = (out_v0, out_v1)
        sems = (so0, so1)

        for h in range(2):
            pltpu.sync_copy(w_hbm.at[b, ct, :, pl.ds(4 * h, 4), :], table_v)

            for q in range(nq):
                pltpu.sync_copy(
                    idx_hbm.at[pl.ds((b * s_dim + q * sq) * 16, 16 * sq)],
                    idx_v)

                def pair(t, carry, q=q, h=h):
                    for p in range(2):
                        sl = 2 * t + p
                        s = q * sq + sl

                        @pl.when(jnp.logical_or(sl >= 2, (q + h) > 0))
                        def _():
                            pltpu.make_async_copy(
                                out_bufs[p],
                                out_hbm.at[b, s, ct, :, pl.ds(4 * h, 4), :],
                                sems[p]).wait()

                        @plsc.parallel_loop(0, 16, unroll=2)
                        def ptile(pt, p=p, sl=sl):
                            for gg in range(8):
                                n = idx_v[sl * 16 + pt, pl.ds(gg * 16, 16)]
                                nt = lax.shift_right_logical(n, 7)
                                nj = lax.bitwise_and(n, 127)
                                for ci_ in range(4):
                                    ci_arr = jnp.full((16,), ci_, jnp.int32)
                                    vals = plsc.load_gather(
                                        table_v, [nt, ci_arr, nj])
                                    out_bufs[p][
                                        pt, ci_, pl.ds(gg * 16, 16)] = vals
                        pltpu.async_copy(
                            out_bufs[p],
                            out_hbm.at[b, s, ct, :, pl.ds(4 * h, 4), :],
                            sems[p])
                    return carry

                lax.fori_loop(0, sq // 2, pair, 0)

        for p in range(2):
            pltpu.make_async_copy(
                out_bufs[p],
                out_hbm.at[b, 0, ct, :, pl.ds(4, 4), :],
                sems[p]).wait()

    return gather_kernel(w, idx2d)


def kernel(inputs):
    b, n, c = inputs.shape
    key = jax.random.key(42)
    k1, k2 = jax.random.split(key)
    group_indices = jax.random.randint(
        k2, (b, _NPOINTS, _NSAMPLE), 0, n, dtype=jnp.int32)
    idx2d = group_indices.transpose(0, 2, 1).reshape(-1, 128)
    w = inputs.transpose(0, 2, 1).reshape(b, 2, 8, 128, 128)
    w = w.transpose(0, 1, 3, 2, 4)
    o6 = _sc_gather_tiled(w, idx2d, b, _NSAMPLE)
    out = o6.transpose(0, 1, 2, 4, 3, 5).reshape(b, _NSAMPLE, c, _NPOINTS)
    return out.transpose(0, 3, 1, 2)


# speedup vs baseline: 124.9367x; 1.0201x over previous
import functools

import jax
import jax.numpy as jnp
from jax import lax
from jax.experimental import pallas as pl
from jax.experimental.pallas import tpu as pltpu
from jax.experimental.pallas import tpu_sc as plsc

_NPOINTS = 2048
_NSAMPLE = 32


def _sc_gather_tiled(w, idx2d, b_dim, s_dim):
    mesh = plsc.VectorSubcoreMesh(core_axis_name="c", subcore_axis_name="s")
    info = plsc.get_sparse_core_info()

    sq = 16
    nq = s_dim // sq

    @functools.partial(
        pl.kernel,
        out_type=jax.ShapeDtypeStruct((b_dim, s_dim, 2, 16, 8, 128), w.dtype),
        mesh=mesh,
        scratch_types=[
            pltpu.VMEM((128, 4, 128), jnp.float32),
            pltpu.VMEM((16 * sq, 128), jnp.int32),
            pltpu.VMEM((16, 4, 128), jnp.float32),
            pltpu.VMEM((16, 4, 128), jnp.float32),
            pltpu.SemaphoreType.DMA,
            pltpu.SemaphoreType.DMA,
        ],
        compiler_params=pltpu.CompilerParams(
            use_tc_tiling_on_sc=False, needs_layout_passes=False),
    )
    def gather_kernel(w_hbm, idx_hbm, out_hbm, table_v, idx_v,
                      out_v0, out_v1, so0, so1):
        wid = lax.axis_index("s") * info.num_cores + lax.axis_index("c")
        b = wid // 2
        ct = wid % 2
        out_bufs = (out_v0, out_v1)
        sems = (so0, so1)

        for h in range(2):
            pltpu.sync_copy(w_hbm.at[b, ct, :, pl.ds(4 * h, 4), :], table_v)

            for q in range(nq):
                pltpu.sync_copy(
                    idx_hbm.at[pl.ds((b * s_dim + q * sq) * 16, 16 * sq)],
                    idx_v)

                def pair(t, carry, q=q, h=h):
                    for p in range(2):
                        sl = 2 * t + p
                        s = q * sq + sl

                        @pl.when(jnp.logical_or(sl >= 2, (q + h) > 0))
                        def _():
                            pltpu.make_async_copy(
                                out_bufs[p],
                                out_hbm.at[b, s, ct, :, pl.ds(4 * h, 4), :],
                                sems[p]).wait()

                        @plsc.parallel_loop(0, 16, unroll=4)
                        def ptile(pt, p=p, sl=sl):
                            for gg in range(8):
                                n = idx_v[sl * 16 + pt, pl.ds(gg * 16, 16)]
                                nt = lax.shift_right_logical(n, 7)
                                nj = lax.bitwise_and(n, 127)
                                for ci_ in range(4):
                                    ci_arr = jnp.full((16,), ci_, jnp.int32)
                                    vals = plsc.load_gather(
                                        table_v, [nt, ci_arr, nj])
                                    out_bufs[p][
                                        pt, ci_, pl.ds(gg * 16, 16)] = vals
                        pltpu.async_copy(
                            out_bufs[p],
                            out_hbm.at[b, s, ct, :, pl.ds(4 * h, 4), :],
                            sems[p])
                    return carry

                lax.fori_loop(0, sq // 2, pair, 0)

        for p in range(2):
            pltpu.make_async_copy(
                out_bufs[p],
                out_hbm.at[b, 0, ct, :, pl.ds(4, 4), :],
                sems[p]).wait()

    return gather_kernel(w, idx2d)


def kernel(inputs):
    b, n, c = inputs.shape
    key = jax.random.key(42)
    k1, k2 = jax.random.split(key)
    group_indices = jax.random.randint(
        k2, (b, _NPOINTS, _NSAMPLE), 0, n, dtype=jnp.int32)
    idx2d = group_indices.transpose(0, 2, 1).reshape(-1, 128)
    w = inputs.transpose(0, 2, 1).reshape(b, 2, 8, 128, 128)
    w = w.transpose(0, 1, 3, 2, 4)
    o6 = _sc_gather_tiled(w, idx2d, b, _NSAMPLE)
    out = o6.transpose(0, 1, 2, 4, 3, 5).reshape(b, _NSAMPLE, c, _NPOINTS)
    return out.transpose(0, 3, 1, 2)
